# Initial kernel scaffold; baseline (speedup 1.0000x reference)
import functools

import jax
import jax.numpy as jnp
from jax import lax
from jax.experimental import pallas as pl
from jax.experimental.pallas import tpu as pltpu
from jax.experimental.pallas import tpu_sc as plsc

N_NODES = 10000
D = 256
DH = 128
E = 160000
NS = 16
E_PAD = 163840
EDGES_PER_SUB = E_PAD // NS
SUB_CHUNK = 128
N_SUB = 4
CHUNK = SUB_CHUNK * N_SUB
N_CHUNKS = EDGES_PER_SUB // CHUNK
ACC_ROWS = N_NODES + 48
ROWS_PER_SUB = N_NODES // NS
ROW_BLK = 500


def _dense_body(x_ref, wc_ref, wl_ref, b_ref, h_ref, base_ref):
    xb = x_ref[...]
    dn = (((1,), (1,)), ((), ()))
    h = lax.dot_general(xb, wc_ref[...], dn, preferred_element_type=jnp.float32)
    base = lax.dot_general(xb, wl_ref[...], dn, preferred_element_type=jnp.float32)
    base = base + b_ref[...]
    h_ref[0] = h[:, :DH]
    h_ref[1] = h[:, DH:]
    base_ref[0] = base[:, :DH]
    base_ref[1] = base[:, DH:]


def _dense(x, W_conv, W_lin, b_lin):
    return pl.pallas_call(
        _dense_body,
        grid=(N_NODES // ROW_BLK,),
        in_specs=[
            pl.BlockSpec((ROW_BLK, D), lambda i: (i, 0)),
            pl.BlockSpec((D, D), lambda i: (0, 0)),
            pl.BlockSpec((D, D), lambda i: (0, 0)),
            pl.BlockSpec((1, D), lambda i: (0, 0)),
        ],
        out_specs=[
            pl.BlockSpec((2, ROW_BLK, DH), lambda i: (0, i, 0)),
            pl.BlockSpec((2, ROW_BLK, DH), lambda i: (0, i, 0)),
        ],
        out_shape=[
            jax.ShapeDtypeStruct((2, N_NODES, DH), jnp.float32),
            jax.ShapeDtypeStruct((2, N_NODES, DH), jnp.float32),
        ],
    )(x, W_conv, W_lin, b_lin.reshape(1, D))


_sc_mesh = plsc.VectorSubcoreMesh(core_axis_name="c", subcore_axis_name="s")


@functools.partial(
    pl.kernel,
    out_type=jax.ShapeDtypeStruct((N_NODES, D), jnp.float32),
    mesh=_sc_mesh,
    scratch_types=[
        pltpu.VMEM((N_SUB, SUB_CHUNK), jnp.int32),
        pltpu.VMEM((N_SUB, SUB_CHUNK), jnp.int32),
        pltpu.VMEM((CHUNK, DH), jnp.float32),
        pltpu.VMEM_SHARED((ACC_ROWS, DH), jnp.float32),
        pltpu.SemaphoreType.DMA,
    ],
)
def _sc_agg(h_hbm, base_hbm, src_hbm, dst_hbm, out_hbm,
            src_v, dst_v, rows_v, acc_sh, sem):
    c = lax.axis_index("c")
    s = lax.axis_index("s")
    r0 = s * ROWS_PER_SUB

    pltpu.sync_copy(base_hbm.at[c, pl.ds(r0, ROWS_PER_SUB)],
                    acc_sh.at[pl.ds(r0, ROWS_PER_SUB)])
    plsc.subcore_barrier()

    idx_row0 = s * (EDGES_PER_SUB // SUB_CHUNK)

    def chunk_body(k, carry):
        r = idx_row0 + k * N_SUB
        pltpu.sync_copy(src_hbm.at[pl.ds(r, N_SUB)], src_v)
        pltpu.sync_copy(dst_hbm.at[pl.ds(r, N_SUB)], dst_v)
        descs = []
        for j in range(N_SUB):
            descs.append(pltpu.async_copy(
                h_hbm.at[c].at[src_v.at[j]],
                rows_v.at[pl.ds(j * SUB_CHUNK, SUB_CHUNK)], sem))
        for d in descs:
            d.wait()
        for j in range(N_SUB):
            pltpu.sync_copy(rows_v.at[pl.ds(j * SUB_CHUNK, SUB_CHUNK)],
                            acc_sh.at[dst_v.at[j]], add=True)
        return carry

    lax.fori_loop(0, N_CHUNKS, chunk_body, 0)
    plsc.subcore_barrier()

    pltpu.sync_copy(acc_sh.at[pl.ds(r0, ROWS_PER_SUB)],
                    out_hbm.at[pl.ds(r0, ROWS_PER_SUB), pl.ds(c * DH, DH)])


def kernel(x, edge_index, W_conv, W_lin, b_lin):
    h2, base2 = _dense(x, W_conv, W_lin, b_lin)
    src = edge_index[0].astype(jnp.int32)
    dst = edge_index[1].astype(jnp.int32)
    pad = E_PAD - E
    src_p = jnp.concatenate([src, jnp.zeros((pad,), jnp.int32)])
    dst_p = jnp.concatenate([dst, jnp.full((pad,), N_NODES, jnp.int32)])
    src2d = src_p.reshape(E_PAD // SUB_CHUNK, SUB_CHUNK)
    dst2d = dst_p.reshape(E_PAD // SUB_CHUNK, SUB_CHUNK)
    return _sc_agg(h2, base2, src2d, dst2d)


# speedup vs baseline: 3.4007x; 3.4007x over previous
import functools

import jax
import jax.numpy as jnp
from jax import lax
from jax.experimental import pallas as pl
from jax.experimental.pallas import tpu as pltpu
from jax.experimental.pallas import tpu_sc as plsc

N_NODES = 10000
D = 256
DH = 128
E = 160000
NS = 16
E_PAD = 163840
EDGES_PER_SUB = E_PAD // NS
SUB_CHUNK = 128
IDX_ROWS = 8
N_SUB = 2
CHUNK = SUB_CHUNK * IDX_ROWS
N_CHUNKS = EDGES_PER_SUB // CHUNK
ACC_ROWS = N_NODES + 48
OUT_ROWS = 640
OUT_ROWS_LAST = N_NODES - (NS - 1) * OUT_ROWS
ROW_BLK = 1000


def _dense_body(x_ref, wc_ref, wl_ref, b_ref, h_ref, base_ref):
    xb = x_ref[...]
    dn = (((1,), (1,)), ((), ()))
    h = lax.dot_general(xb, wc_ref[...], dn, preferred_element_type=jnp.float32)
    base = lax.dot_general(xb, wl_ref[...], dn, preferred_element_type=jnp.float32)
    base = base + b_ref[...]
    h_ref[0] = h[:, :DH]
    h_ref[1] = h[:, DH:]
    base_ref[0] = base[:, :DH]
    base_ref[1] = base[:, DH:]


def _dense(x, W_conv, W_lin, b_lin):
    return pl.pallas_call(
        _dense_body,
        grid=(N_NODES // ROW_BLK,),
        in_specs=[
            pl.BlockSpec((ROW_BLK, D), lambda i: (i, 0)),
            pl.BlockSpec((D, D), lambda i: (0, 0)),
            pl.BlockSpec((D, D), lambda i: (0, 0)),
            pl.BlockSpec((1, D), lambda i: (0, 0)),
        ],
        out_specs=[
            pl.BlockSpec((2, ROW_BLK, DH), lambda i: (0, i, 0)),
            pl.BlockSpec((2, ROW_BLK, DH), lambda i: (0, i, 0)),
        ],
        out_shape=[
            jax.ShapeDtypeStruct((2, N_NODES, DH), jnp.float32),
            jax.ShapeDtypeStruct((2, N_NODES, DH), jnp.float32),
        ],
    )(x, W_conv, W_lin, b_lin.reshape(1, D))


_sc_mesh = plsc.VectorSubcoreMesh(core_axis_name="c", subcore_axis_name="s")


@functools.partial(
    pl.kernel,
    out_type=jax.ShapeDtypeStruct((N_NODES, D), jnp.float32),
    mesh=_sc_mesh,
    scratch_types=[
        pltpu.VMEM((IDX_ROWS, SUB_CHUNK), jnp.int32),
        pltpu.VMEM((IDX_ROWS, SUB_CHUNK), jnp.int32),
        pltpu.VMEM((N_SUB * SUB_CHUNK, DH), jnp.float32),
        pltpu.VMEM_SHARED((ACC_ROWS, DH), jnp.float32),
        pltpu.SemaphoreType.DMA,
    ],
)
def _sc_agg(h_hbm, base_hbm, src_hbm, dst_hbm, out_hbm,
            src_v, dst_v, rows_v, acc_sh, sem):
    c = lax.axis_index("c")
    s = lax.axis_index("s")
    r0 = s * OUT_ROWS

    @pl.when(s < NS - 1)
    def _():
        pltpu.sync_copy(base_hbm.at[c, pl.ds(r0, OUT_ROWS)],
                        acc_sh.at[pl.ds(r0, OUT_ROWS)])

    @pl.when(s == NS - 1)
    def _():
        pltpu.sync_copy(base_hbm.at[c, pl.ds((NS - 1) * OUT_ROWS, OUT_ROWS_LAST)],
                        acc_sh.at[pl.ds((NS - 1) * OUT_ROWS, OUT_ROWS_LAST)])

    plsc.subcore_barrier()

    idx_row0 = s * (EDGES_PER_SUB // SUB_CHUNK)

    def chunk_body(k, carry):
        r = idx_row0 + k * IDX_ROWS
        pltpu.sync_copy(src_hbm.at[pl.ds(r, IDX_ROWS)], src_v)
        pltpu.sync_copy(dst_hbm.at[pl.ds(r, IDX_ROWS)], dst_v)
        for half in range(IDX_ROWS // N_SUB):
            descs = []
            for j in range(N_SUB):
                descs.append(pltpu.async_copy(
                    h_hbm.at[c].at[src_v.at[half * N_SUB + j]],
                    rows_v.at[pl.ds(j * SUB_CHUNK, SUB_CHUNK)], sem))
            for d in descs:
                d.wait()
            for j in range(N_SUB):
                pltpu.sync_copy(rows_v.at[pl.ds(j * SUB_CHUNK, SUB_CHUNK)],
                                acc_sh.at[dst_v.at[half * N_SUB + j]], add=True)
        return carry

    lax.fori_loop(0, N_CHUNKS, chunk_body, 0)
    plsc.subcore_barrier()

    @pl.when(s < NS - 1)
    def _():
        pltpu.sync_copy(acc_sh.at[pl.ds(r0, OUT_ROWS)],
                        out_hbm.at[pl.ds(r0, OUT_ROWS), pl.ds(c * DH, DH)])

    @pl.when(s == NS - 1)
    def _():
        pltpu.sync_copy(
            acc_sh.at[pl.ds((NS - 1) * OUT_ROWS, OUT_ROWS_LAST)],
            out_hbm.at[pl.ds((NS - 1) * OUT_ROWS, OUT_ROWS_LAST),
                       pl.ds(c * DH, DH)])


def kernel(x, edge_index, W_conv, W_lin, b_lin):
    h2, base2 = _dense(x, W_conv, W_lin, b_lin)
    src = edge_index[0].astype(jnp.int32)
    dst = edge_index[1].astype(jnp.int32)
    pad = E_PAD - E
    src_p = jnp.concatenate([src, jnp.zeros((pad,), jnp.int32)])
    dst_p = jnp.concatenate([dst, jnp.full((pad,), N_NODES, jnp.int32)])
    src2d = src_p.reshape(E_PAD // SUB_CHUNK, SUB_CHUNK)
    dst2d = dst_p.reshape(E_PAD // SUB_CHUNK, SUB_CHUNK)
    return _sc_agg(h2, base2, src2d, dst2d)


# speedup vs baseline: 3.7144x; 1.0923x over previous
import functools

import jax
import jax.numpy as jnp
from jax import lax
from jax.experimental import pallas as pl
from jax.experimental.pallas import tpu as pltpu
from jax.experimental.pallas import tpu_sc as plsc

N_NODES = 10000
D = 256
DH = 128
E = 160000
NS = 16
E_PAD = 163840
EDGES_PER_SUB = E_PAD // NS
SUB_CHUNK = 128
IDX_ROWS = 8
N_SUB = 2
CHUNK = SUB_CHUNK * IDX_ROWS
N_CHUNKS = EDGES_PER_SUB // CHUNK
ACC_ROWS = N_NODES + 48
OUT_ROWS = 640
OUT_ROWS_LAST = N_NODES - (NS - 1) * OUT_ROWS
ROW_BLK = 1000


def _dense_body(x_ref, wc_ref, wl_ref, b_ref, h_ref, base_ref):
    xb = x_ref[...]
    dn = (((1,), (1,)), ((), ()))
    h = lax.dot_general(xb, wc_ref[...], dn, preferred_element_type=jnp.float32)
    base = lax.dot_general(xb, wl_ref[...], dn, preferred_element_type=jnp.float32)
    base = base + b_ref[...]
    h_ref[0] = h[:, :DH]
    h_ref[1] = h[:, DH:]
    base_ref[0] = base[:, :DH]
    base_ref[1] = base[:, DH:]


def _dense(x, W_conv, W_lin, b_lin):
    return pl.pallas_call(
        _dense_body,
        grid=(N_NODES // ROW_BLK,),
        in_specs=[
            pl.BlockSpec((ROW_BLK, D), lambda i: (i, 0)),
            pl.BlockSpec((D, D), lambda i: (0, 0)),
            pl.BlockSpec((D, D), lambda i: (0, 0)),
            pl.BlockSpec((1, D), lambda i: (0, 0)),
        ],
        out_specs=[
            pl.BlockSpec((2, ROW_BLK, DH), lambda i: (0, i, 0)),
            pl.BlockSpec((2, ROW_BLK, DH), lambda i: (0, i, 0)),
        ],
        out_shape=[
            jax.ShapeDtypeStruct((2, N_NODES, DH), jnp.float32),
            jax.ShapeDtypeStruct((2, N_NODES, DH), jnp.float32),
        ],
    )(x, W_conv, W_lin, b_lin.reshape(1, D))


_sc_mesh = plsc.VectorSubcoreMesh(core_axis_name="c", subcore_axis_name="s")


@functools.partial(
    pl.kernel,
    out_type=jax.ShapeDtypeStruct((N_NODES, D), jnp.float32),
    mesh=_sc_mesh,
    scratch_types=[
        pltpu.VMEM((IDX_ROWS, SUB_CHUNK), jnp.int32),
        pltpu.VMEM((IDX_ROWS, SUB_CHUNK), jnp.int32),
        pltpu.VMEM((SUB_CHUNK, DH), jnp.float32),
        pltpu.VMEM((SUB_CHUNK, DH), jnp.float32),
        pltpu.VMEM_SHARED((ACC_ROWS, DH), jnp.float32),
        pltpu.SemaphoreType.DMA,
        pltpu.SemaphoreType.DMA,
        pltpu.SemaphoreType.DMA,
        pltpu.SemaphoreType.DMA,
    ],
)
def _sc_agg(h_hbm, base_hbm, src_hbm, dst_hbm, out_hbm,
            src_v, dst_v, rows_a, rows_b, acc_sh,
            gsem_a, gsem_b, ssem_a, ssem_b):
    c = lax.axis_index("c")
    s = lax.axis_index("s")
    r0 = s * OUT_ROWS

    @pl.when(s < NS - 1)
    def _():
        pltpu.sync_copy(base_hbm.at[c, pl.ds(r0, OUT_ROWS)],
                        acc_sh.at[pl.ds(r0, OUT_ROWS)])

    @pl.when(s == NS - 1)
    def _():
        pltpu.sync_copy(base_hbm.at[c, pl.ds((NS - 1) * OUT_ROWS, OUT_ROWS_LAST)],
                        acc_sh.at[pl.ds((NS - 1) * OUT_ROWS, OUT_ROWS_LAST)])

    plsc.subcore_barrier()

    idx_row0 = s * (EDGES_PER_SUB // SUB_CHUNK)

    bufs = (rows_a, rows_b)
    gsems = (gsem_a, gsem_b)
    ssems = (ssem_a, ssem_b)

    def chunk_body(k, carry):
        r = idx_row0 + k * IDX_ROWS
        pltpu.sync_copy(src_hbm.at[pl.ds(r, IDX_ROWS)], src_v)
        pltpu.sync_copy(dst_hbm.at[pl.ds(r, IDX_ROWS)], dst_v)
        g = [None] * IDX_ROWS
        sc = [None] * IDX_ROWS
        g[0] = pltpu.async_copy(h_hbm.at[c].at[src_v.at[0]], bufs[0], gsems[0])
        for j in range(IDX_ROWS):
            if j + 1 < IDX_ROWS:
                if j >= 1:
                    sc[j - 1].wait()
                g[j + 1] = pltpu.async_copy(
                    h_hbm.at[c].at[src_v.at[j + 1]],
                    bufs[(j + 1) % 2], gsems[(j + 1) % 2])
            g[j].wait()
            sc[j] = pltpu.async_copy(bufs[j % 2], acc_sh.at[dst_v.at[j]],
                                     ssems[j % 2], add=True)
        sc[IDX_ROWS - 2].wait()
        sc[IDX_ROWS - 1].wait()
        return carry

    lax.fori_loop(0, N_CHUNKS, chunk_body, 0)
    plsc.subcore_barrier()

    @pl.when(s < NS - 1)
    def _():
        pltpu.sync_copy(acc_sh.at[pl.ds(r0, OUT_ROWS)],
                        out_hbm.at[pl.ds(r0, OUT_ROWS), pl.ds(c * DH, DH)])

    @pl.when(s == NS - 1)
    def _():
        pltpu.sync_copy(
            acc_sh.at[pl.ds((NS - 1) * OUT_ROWS, OUT_ROWS_LAST)],
            out_hbm.at[pl.ds((NS - 1) * OUT_ROWS, OUT_ROWS_LAST),
                       pl.ds(c * DH, DH)])


def kernel(x, edge_index, W_conv, W_lin, b_lin):
    h2, base2 = _dense(x, W_conv, W_lin, b_lin)
    src = edge_index[0].astype(jnp.int32)
    dst = edge_index[1].astype(jnp.int32)
    pad = E_PAD - E
    src_p = jnp.concatenate([src, jnp.zeros((pad,), jnp.int32)])
    dst_p = jnp.concatenate([dst, jnp.full((pad,), N_NODES, jnp.int32)])
    src2d = src_p.reshape(E_PAD // SUB_CHUNK, SUB_CHUNK)
    dst2d = dst_p.reshape(E_PAD // SUB_CHUNK, SUB_CHUNK)
    return _sc_agg(h2, base2, src2d, dst2d)
